# baseline (speedup 1.0000x reference)
import functools

import jax
import jax.numpy as jnp
from jax import lax
from jax.experimental import pallas as pl
from jax.experimental.pallas import tpu as pltpu
from jax.experimental.pallas import tpu_sc as plsc


@functools.cache
def _build(V, D, R, S):
    info = plsc.get_sparse_core_info()
    NC, NS = info.num_cores, info.num_subcores
    NW = NC * NS
    assert R % (NW * 128) == 0
    CB = R // NW
    P = CB // 128
    TS = -(-S // 8)
    assert S % 2 == 0

    mesh = plsc.VectorSubcoreMesh(core_axis_name="c", subcore_axis_name="s")

    @functools.partial(
        pl.kernel,
        mesh=mesh,
        compiler_params=pltpu.CompilerParams(use_tc_tiling_on_sc=True),
        out_type=jax.ShapeDtypeStruct((S, R // 128, 128), jnp.int32),
        name="sc_idx_detile",
        scratch_types=[
            pltpu.VMEM((8, P, 128), jnp.int32),
        ],
    )
    def detile_kernel(idxt_hbm, out_hbm, vm):
        wid = lax.axis_index("s") * NC + lax.axis_index("c")
        b0 = wid * CB

        def body(ts, carry):
            for p in range(P):
                pltpu.sync_copy(
                    idxt_hbm.at[pl.ds(ts * 8, 8),
                                pl.ds(b0 + p * 128, 128)],
                    vm.at[:, p])
            for r in range(8):
                @pl.when(ts * 8 + r < S)
                def _():
                    pltpu.sync_copy(
                        vm.at[r],
                        out_hbm.at[ts * 8 + r, pl.ds(wid * P, P), :])
            return carry

        lax.fori_loop(0, TS, body, 0)

    @functools.partial(
        pl.kernel,
        mesh=mesh,
        compiler_params=pltpu.CompilerParams(use_tc_tiling_on_sc=False),
        out_type=jax.ShapeDtypeStruct((R, S, D), jnp.float32),
        name="sc_embedding_gather",
        scratch_types=[
            pltpu.VMEM((P, 128), jnp.int32),
            pltpu.VMEM((P, 128), jnp.int32),
            pltpu.VMEM((CB, D), jnp.float32),
            pltpu.VMEM((CB, D), jnp.float32),
            pltpu.SemaphoreType.DMA,
            pltpu.SemaphoreType.DMA,
            pltpu.SemaphoreType.DMA,
            pltpu.SemaphoreType.DMA,
        ],
    )
    def gather_kernel(idx3_hbm, table_hbm, out_hbm, idx0, idx1, rows0, rows1,
                      gsem0, gsem1, ssem0, ssem1):
        wid = lax.axis_index("s") * NC + lax.axis_index("c")
        b0 = wid * CB

        def load_idx(s, idx_v):
            pltpu.sync_copy(idx3_hbm.at[s, pl.ds(wid * P, P), :], idx_v)

        def g_desc(idx_v, rows, gsem, p):
            return pltpu.make_async_copy(
                table_hbm.at[idx_v.at[p]],
                rows.at[pl.ds(p * 128, 128), :], gsem)

        def g_start(idx_v, rows, gsem):
            for p in range(P):
                g_desc(idx_v, rows, gsem, p).start()

        def g_wait(idx_v, rows, gsem):
            for p in range(P):
                g_desc(idx_v, rows, gsem, p).wait()

        def s_desc(s, rows, ssem):
            return pltpu.make_async_copy(
                rows, out_hbm.at[pl.ds(b0, CB), s, :], ssem)

        load_idx(0, idx0)
        g_start(idx0, rows0, gsem0)
        load_idx(1, idx1)
        g_start(idx1, rows1, gsem1)

        bufs = ((idx0, rows0, gsem0, ssem0), (idx1, rows1, gsem1, ssem1))

        def body(g2, carry):
            g = g2 * 2
            for b in range(2):
                s = g + b
                idx_v, rows, gsem, ssem = bufs[b]
                g_wait(idx_v, rows, gsem)
                s_desc(s, rows, ssem).start()

                @pl.when(s + 2 < S)
                def _():
                    s_desc(s, rows, ssem).wait()
                    load_idx(s + 2, idx_v)
                    g_start(idx_v, rows, gsem)

            return carry

        lax.fori_loop(0, S // 2, body, 0)
        s_desc(S - 2, rows0, ssem0).wait()
        s_desc(S - 1, rows1, ssem1).wait()

    def run(data, in_table):
        idxt = data.astype(jnp.int32).T
        idx3 = detile_kernel(idxt)
        return gather_kernel(idx3, in_table)

    return run


def kernel(data, in_table, out_table):
    R, S = data.shape
    V, D = in_table.shape
    return _build(V, D, R, S)(data, in_table)


# speedup vs baseline: 1.0106x; 1.0106x over previous
import functools

import jax
import jax.numpy as jnp
from jax import lax
from jax.experimental import pallas as pl
from jax.experimental.pallas import tpu as pltpu
from jax.experimental.pallas import tpu_sc as plsc


@functools.cache
def _build(V, D, B):
    info = plsc.get_sparse_core_info()
    NC, NS = info.num_cores, info.num_subcores
    NW = NC * NS
    assert B % NW == 0
    b_per_w = B // NW
    C = 640
    assert b_per_w % (2 * C) == 0
    n_chunks = b_per_w // C

    mesh = plsc.VectorSubcoreMesh(core_axis_name="c", subcore_axis_name="s")

    @functools.partial(
        pl.kernel,
        mesh=mesh,
        compiler_params=pltpu.CompilerParams(use_tc_tiling_on_sc=False),
        out_type=jax.ShapeDtypeStruct((B, D), jnp.float32),
        name="sc_embedding_gather",
        scratch_types=[
            pltpu.VMEM((b_per_w,), jnp.int32),
            pltpu.VMEM((C, D), jnp.float32),
            pltpu.VMEM((C, D), jnp.float32),
            pltpu.SemaphoreType.DMA,
            pltpu.SemaphoreType.DMA,
            pltpu.SemaphoreType.DMA,
            pltpu.SemaphoreType.DMA,
        ],
    )
    def gather_kernel(idx_hbm, table_hbm, out_hbm, idx_v, rows0, rows1,
                      gsem0, gsem1, ssem0, ssem1):
        wid = lax.axis_index("s") * NC + lax.axis_index("c")
        base = wid * b_per_w
        pltpu.sync_copy(idx_hbm.at[pl.ds(base, b_per_w)], idx_v)

        def g_desc(c, rows, gsem):
            return pltpu.make_async_copy(
                table_hbm.at[idx_v.at[pl.ds(c * C, C)]], rows, gsem)

        def s_desc(c, rows, ssem):
            return pltpu.make_async_copy(
                rows, out_hbm.at[pl.ds(base + c * C, C)], ssem)

        g_desc(0, rows0, gsem0).start()
        g_desc(1, rows1, gsem1).start()

        bufs = ((rows0, gsem0, ssem0), (rows1, gsem1, ssem1))

        def body(g2, carry):
            g = g2 * 2
            for b in range(2):
                c = g + b
                rows, gsem, ssem = bufs[b]
                g_desc(c, rows, gsem).wait()
                s_desc(c, rows, ssem).start()

                @pl.when(c + 2 < n_chunks)
                def _():
                    s_desc(c, rows, ssem).wait()
                    g_desc(c + 2, rows, gsem).start()

            return carry

        lax.fori_loop(0, n_chunks // 2, body, 0)
        s_desc(n_chunks - 2, rows0, ssem0).wait()
        s_desc(n_chunks - 1, rows1, ssem1).wait()

    return gather_kernel


def kernel(data, in_table, out_table):
    R, S = data.shape
    V, D = in_table.shape
    idx = data.reshape(R * S).astype(jnp.int32)
    out = _build(V, D, R * S)(idx, in_table)
    return out.reshape(R, S, D)
